# baseline (speedup 1.0000x reference)
import functools
import jax
import jax.numpy as jnp
from jax import lax
from jax.experimental import pallas as pl
from jax.experimental.pallas import tpu as pltpu
from jax.experimental.pallas import tpu_sc as plsc

NP_ = 8
N_ = 500
NPAD = 512
BIG = 1e30
L = 16
NC = 2
NS = 16
NW = NC * NS


def _lrelu(t):
    return jnp.where(t >= 0, t, 0.01 * t)


def _unlace(x):
    return x.reshape(NW, NP_, L).transpose(0, 2, 1).reshape(NPAD, NP_)


def _sqrt16(x):
    x = jnp.maximum(x, 0.0)
    i = lax.bitcast_convert_type(x, jnp.int32)
    i = jnp.int32(0x5F3759DF) - lax.shift_right_arithmetic(i, 1)
    y = lax.bitcast_convert_type(i, jnp.float32)
    for _ in range(3):
        y = y * (1.5 - 0.5 * x * y * y)
    return x * y


def _sc_body(px_h, py_h, pz_h, tx_h, ty_h, tz_h, cx_h, cy_h, cz_h,
             w1_h, i1_h, w2_h, i2_h,
             px_v, py_v, pz_v, tx_v, ty_v, tz_v,
             qpx, qpy, qpz, qtx, qty, qtz, qcx, qcy, qcz,
             w1b, i1b, w2b, i2b):
    wid = lax.axis_index("s") * NC + lax.axis_index("c")
    base = wid * L

    pltpu.sync_copy(px_h, px_v)
    pltpu.sync_copy(py_h, py_v)
    pltpu.sync_copy(pz_h, pz_v)
    pltpu.sync_copy(tx_h, tx_v)
    pltpu.sync_copy(ty_h, ty_v)
    pltpu.sync_copy(tz_h, tz_v)
    pltpu.sync_copy(px_h.at[pl.ds(base, L)], qpx)
    pltpu.sync_copy(py_h.at[pl.ds(base, L)], qpy)
    pltpu.sync_copy(pz_h.at[pl.ds(base, L)], qpz)
    pltpu.sync_copy(tx_h.at[pl.ds(base, L)], qtx)
    pltpu.sync_copy(ty_h.at[pl.ds(base, L)], qty)
    pltpu.sync_copy(tz_h.at[pl.ds(base, L)], qtz)
    pltpu.sync_copy(cx_h.at[pl.ds(base, L)], qcx)
    pltpu.sync_copy(cy_h.at[pl.ds(base, L)], qcy)
    pltpu.sync_copy(cz_h.at[pl.ds(base, L)], qcz)

    apx, apy, apz = qpx[...], qpy[...], qpz[...]
    atx, aty, atz = qtx[...], qty[...], qtz[...]
    acx, acy, acz = qcx[...], qcy[...], qcz[...]

    def insert(vs, js, c, ci, ps=None, cp=None):
        vs, js = list(vs), list(js)
        ps = None if ps is None else list(ps)
        for t in range(NP_):
            lt = c < vs[t]
            nv = jnp.where(lt, c, vs[t])
            nc = jnp.where(lt, vs[t], c)
            ni = jnp.where(lt, ci, js[t])
            nci = jnp.where(lt, js[t], ci)
            if ps is not None:
                np_ = jnp.where(lt, cp, ps[t])
                ncp = jnp.where(lt, ps[t], cp)
                ps[t], cp = np_, ncp
            vs[t], js[t], c, ci = nv, ni, nc, nci
        if ps is None:
            return tuple(vs), tuple(js)
        return tuple(vs), tuple(js), tuple(ps)

    def body(ch, carry):
        v1, i1, p1, v2, i2 = carry
        off = ch * L
        tvx = tx_v[pl.ds(off, L)]
        tvy = ty_v[pl.ds(off, L)]
        tvz = tz_v[pl.ds(off, L)]
        pvx = px_v[pl.ds(off, L)]
        pvy = py_v[pl.ds(off, L)]
        pvz = pz_v[pl.ds(off, L)]
        for j in range(L):
            rx, ry, rz = tvx[j], tvy[j], tvz[j]
            sx, sy, sz = pvx[j], pvy[j], pvz[j]
            dx, dy, dz = apx - rx, apy - ry, apz - rz
            d1 = dx * dx + dy * dy + dz * dz
            ux, uy, uz = acx - rx, acy - ry, acz - rz
            e1d = ux * ux + uy * uy + uz * uz
            ex, ey, ez = atx - sx, aty - sy, atz - sz
            d2 = ex * ex + ey * ey + ez * ez
            mi = jnp.full((L,), off + j, jnp.int32)
            v1, i1, p1 = insert(v1, i1, d1, mi, p1, e1d)
            v2, i2 = insert(v2, i2, d2, mi)
        return (v1, i1, p1, v2, i2)

    vinit = tuple(jnp.full((L,), BIG, jnp.float32) for _ in range(NP_))
    iinit = tuple(jnp.zeros((L,), jnp.int32) for _ in range(NP_))
    pinit = tuple(jnp.zeros((L,), jnp.float32) for _ in range(NP_))
    v1, i1, p1, v2, i2 = lax.fori_loop(0, NPAD // L, body,
                                       (vinit, iinit, pinit, vinit, iinit))

    lane = lax.iota(jnp.int32, L)
    valid = (lane + base) < N_
    for t in range(NP_):
        e1 = jnp.exp(-_sqrt16(p1[t]))
        e1 = jnp.where(valid, e1, 0.0)
        w1b[pl.ds(t * L, L)] = e1
        i1b[pl.ds(t * L, L)] = i1[t]
        e2 = jnp.exp(-_sqrt16(v2[t]))
        e2 = jnp.where(valid, e2, 0.0)
        w2b[pl.ds(t * L, L)] = e2
        i2b[pl.ds(t * L, L)] = i2[t]

    ob = wid * (L * NP_)
    pltpu.sync_copy(w1b, w1_h.at[pl.ds(ob, L * NP_)])
    pltpu.sync_copy(i1b, i1_h.at[pl.ds(ob, L * NP_)])
    pltpu.sync_copy(w2b, w2_h.at[pl.ds(ob, L * NP_)])
    pltpu.sync_copy(i2b, i2_h.at[pl.ds(ob, L * NP_)])


def _sc_knn(px, py, pz, tx, ty, tz, cx, cy, cz):
    f32, i32 = jnp.float32, jnp.int32
    grp = L * NP_
    run = pl.kernel(
        _sc_body,
        out_type=(
            jax.ShapeDtypeStruct((NW * grp,), f32),
            jax.ShapeDtypeStruct((NW * grp,), i32),
            jax.ShapeDtypeStruct((NW * grp,), f32),
            jax.ShapeDtypeStruct((NW * grp,), i32),
        ),
        mesh=plsc.VectorSubcoreMesh(core_axis_name="c", subcore_axis_name="s"),
        scratch_types=(
            [pltpu.VMEM((NPAD,), f32) for _ in range(6)]
            + [pltpu.VMEM((L,), f32) for _ in range(9)]
            + [pltpu.VMEM((grp,), f32), pltpu.VMEM((grp,), i32),
               pltpu.VMEM((grp,), f32), pltpu.VMEM((grp,), i32)]
        ),
    )
    return run(px, py, pz, tx, ty, tz, cx, cy, cz)


def _mlp_body(Pr, Gr, Wp1t, bp1, Wp2t, bp2, W1t, b1, W2t, b2,
              Wps1t, bps1, Wps2t, bps2, cf_ref, sfull_ref, spfull_ref):
    dot = lambda a, b: jnp.dot(a, b, preferred_element_type=jnp.float32)

    def mlp2(X, Wat, ba, Wbt, bb):
        return dot(_lrelu(dot(X, Wat[...]) + ba[...]), Wbt[...]) + bb[...]

    cf = mlp2(Pr[...], Wp1t, bp1, Wp2t, bp2)
    cf_ref[...] = cf
    sfull_ref[...] = mlp2(Gr[...], W1t, b1, W2t, b2)
    spfull_ref[...] = mlp2(cf, Wps1t, bps1, Wps2t, bps2)


def _final_body(cf_ref, sfull_ref, spfull_ref, Gr, w1_ref, i1_ref,
                w2_ref, i2_ref, Wf1at, Wf1bt, bf1, Wf2at, Wf2bt, bf2,
                Wfat, Wfbt, bf, out_ref):
    dot = lambda a, b: jnp.dot(a, b, preferred_element_type=jnp.float32)
    col_iota = jax.lax.broadcasted_iota(jnp.int32, (1, NPAD), 1)

    w1 = w1_ref[...]
    w2 = w2_ref[...]
    i1 = i1_ref[...]
    i2 = i2_ref[...]
    s1 = 1.0 / (NP_ * jnp.sum(w1))
    s2 = 1.0 / (NP_ * jnp.sum(w2))

    A1 = jnp.zeros((NPAD, NPAD), jnp.float32)
    A2 = jnp.zeros((NPAD, NPAD), jnp.float32)
    for t in range(NP_):
        A1 = A1 + w1[:, t:t + 1] * (col_iota == i1[:, t:t + 1]).astype(jnp.float32)
        A2 = A2 + w2[:, t:t + 1] * (col_iota == i2[:, t:t + 1]).astype(jnp.float32)

    cf = cf_ref[...]
    G = Gr[...]
    sf = dot(A1, sfull_ref[...]) * s1
    sfp = dot(A2, spfull_ref[...]) * s2

    final1 = dot(sf, Wf1at[...]) + dot(cf, Wf1bt[...]) + bf1[...]
    final2 = dot(sfp, Wf2at[...]) + dot(G, Wf2bt[...]) + bf2[...]
    out_ref[...] = (dot(_lrelu(final2), Wfat[...])
                    + dot(_lrelu(final1), Wfbt[...]) + bf[...])


def kernel(img_feat, cloud, cloud_tar, W1, b1, W2, b2, Wps1, bps1, Wps2, bps2,
           Wp1, bp1, Wp2, bp2, Wf1, bf1, Wf2, bf2, Wf, bf):
    f32 = jnp.float32

    def pad1(v):
        return jnp.pad(v, (0, NPAD - N_), constant_values=1e17).astype(f32)

    def padr(x):
        return jnp.pad(x, ((0, NPAD - x.shape[0]), (0, 0))).astype(f32)

    P3 = cloud[0]
    T3 = cloud_tar[0]
    C2 = cloud.reshape(3, N_)

    w1x, i1x, w2x, i2x = _sc_knn(
        pad1(P3[:, 0]), pad1(P3[:, 1]), pad1(P3[:, 2]),
        pad1(T3[:, 0]), pad1(T3[:, 1]), pad1(T3[:, 2]),
        pad1(C2[0]), pad1(C2[1]), pad1(C2[2]))

    Pr = padr(jnp.pad(P3, ((0, 0), (0, 5))))
    Gr = padr(img_feat[0].T)
    row2 = lambda b: b[None, :].astype(f32)

    mlp_args = (
        Pr, Gr,
        jnp.pad(Wp1.T, ((0, 5), (0, 0))).astype(f32), row2(bp1),
        Wp2.T.astype(f32), row2(bp2),
        W1.T.astype(f32), row2(b1), W2.T.astype(f32), row2(b2),
        Wps1.T.astype(f32), row2(bps1), Wps2.T.astype(f32), row2(bps2),
    )
    shp = jax.ShapeDtypeStruct((NPAD, 128), f32)
    cf, sfull, spfull = pl.pallas_call(
        _mlp_body,
        out_shape=(shp, shp, shp),
        in_specs=[pl.BlockSpec(memory_space=pltpu.VMEM) for _ in mlp_args],
        out_specs=(pl.BlockSpec(memory_space=pltpu.VMEM),) * 3,
    )(*mlp_args)

    fin_args = (
        cf, sfull, spfull, Gr,
        _unlace(w1x), _unlace(i1x), _unlace(w2x), _unlace(i2x),
        Wf1[:, :128].T.astype(f32), Wf1[:, 128:].T.astype(f32), row2(bf1),
        Wf2[:, :128].T.astype(f32), Wf2[:, 128:].T.astype(f32), row2(bf2),
        Wf[:, :64].T.astype(f32), Wf[:, 64:].T.astype(f32), row2(bf),
    )
    out = pl.pallas_call(
        _final_body,
        out_shape=jax.ShapeDtypeStruct((NPAD, 64), f32),
        in_specs=[pl.BlockSpec(memory_space=pltpu.VMEM) for _ in fin_args],
        out_specs=pl.BlockSpec(memory_space=pltpu.VMEM),
    )(*fin_args)

    return out[:N_].T[None]


# speedup vs baseline: 1.1900x; 1.1900x over previous
import jax
import jax.numpy as jnp
from jax import lax
from jax.experimental import pallas as pl
from jax.experimental.pallas import tpu as pltpu
from jax.experimental.pallas import tpu_sc as plsc

NP_ = 8
N_ = 500
NPAD = 512
L = 16
NC = 2
NS = 16
NW = NC * NS
IMASK = 0x1FF
KINIT = 0x7F7FFFFF


def _lrelu(t):
    return jnp.where(t >= 0, t, 0.01 * t)


def _unlace(x):
    return x.reshape(NW, NP_, L).transpose(0, 2, 1).reshape(NPAD, NP_)


def _sc_body(px_h, py_h, pz_h, tx_h, ty_h, tz_h,
             k1_h, k2_h,
             px_v, py_v, pz_v, tx_v, ty_v, tz_v,
             qpx, qpy, qpz, qtx, qty, qtz,
             k1b, k2b):
    wid = lax.axis_index("s") * NC + lax.axis_index("c")
    base = wid * L

    pltpu.sync_copy(px_h, px_v)
    pltpu.sync_copy(py_h, py_v)
    pltpu.sync_copy(pz_h, pz_v)
    pltpu.sync_copy(tx_h, tx_v)
    pltpu.sync_copy(ty_h, ty_v)
    pltpu.sync_copy(tz_h, tz_v)
    pltpu.sync_copy(px_h.at[pl.ds(base, L)], qpx)
    pltpu.sync_copy(py_h.at[pl.ds(base, L)], qpy)
    pltpu.sync_copy(pz_h.at[pl.ds(base, L)], qpz)
    pltpu.sync_copy(tx_h.at[pl.ds(base, L)], qtx)
    pltpu.sync_copy(ty_h.at[pl.ds(base, L)], qty)
    pltpu.sync_copy(tz_h.at[pl.ds(base, L)], qtz)

    apx, apy, apz = qpx[...], qpy[...], qpz[...]
    atx, aty, atz = qtx[...], qty[...], qtz[...]

    def insert(ks, c):
        ks = list(ks)
        for t in range(NP_):
            nk = jnp.minimum(ks[t], c)
            c = jnp.maximum(ks[t], c)
            ks[t] = nk
        return tuple(ks)

    def body(ch, carry):
        k1, k2 = carry
        off = ch * L
        tvx = tx_v[pl.ds(off, L)]
        tvy = ty_v[pl.ds(off, L)]
        tvz = tz_v[pl.ds(off, L)]
        pvx = px_v[pl.ds(off, L)]
        pvy = py_v[pl.ds(off, L)]
        pvz = pz_v[pl.ds(off, L)]
        for j in range(L):
            rx, ry, rz = tvx[j], tvy[j], tvz[j]
            sx, sy, sz = pvx[j], pvy[j], pvz[j]
            dx, dy, dz = apx - rx, apy - ry, apz - rz
            d1 = dx * dx + dy * dy + dz * dz
            ex, ey, ez = atx - sx, aty - sy, atz - sz
            d2 = ex * ex + ey * ey + ez * ez
            m = off + j
            c1 = (lax.bitcast_convert_type(d1, jnp.int32) & ~IMASK) | m
            c2 = (lax.bitcast_convert_type(d2, jnp.int32) & ~IMASK) | m
            k1 = insert(k1, c1)
            k2 = insert(k2, c2)
        return (k1, k2)

    kinit = tuple(jnp.full((L,), KINIT, jnp.int32) for _ in range(NP_))
    k1, k2 = lax.fori_loop(0, NPAD // L, body, (kinit, kinit))

    for t in range(NP_):
        k1b[pl.ds(t * L, L)] = k1[t]
        k2b[pl.ds(t * L, L)] = k2[t]

    ob = wid * (L * NP_)
    pltpu.sync_copy(k1b, k1_h.at[pl.ds(ob, L * NP_)])
    pltpu.sync_copy(k2b, k2_h.at[pl.ds(ob, L * NP_)])


def _sc_knn(px, py, pz, tx, ty, tz):
    i32 = jnp.int32
    grp = L * NP_
    run = pl.kernel(
        _sc_body,
        out_type=(
            jax.ShapeDtypeStruct((NW * grp,), i32),
            jax.ShapeDtypeStruct((NW * grp,), i32),
        ),
        mesh=plsc.VectorSubcoreMesh(core_axis_name="c", subcore_axis_name="s"),
        scratch_types=(
            [pltpu.VMEM((NPAD,), jnp.float32) for _ in range(6)]
            + [pltpu.VMEM((L,), jnp.float32) for _ in range(6)]
            + [pltpu.VMEM((grp,), i32), pltpu.VMEM((grp,), i32)]
        ),
    )
    return run(px, py, pz, tx, ty, tz)


def _tc_body(Pr, Gr, Cr, Tc, k1_ref, k2_ref,
             Wp1t, bp1, Wp2t, bp2, W1t, b1, W2t, b2,
             Wps1t, bps1, Wps2t, bps2,
             Wf1at, Wf1bt, bf1, Wf2at, Wf2bt, bf2,
             Wfat, Wfbt, bf, out_ref):
    dot = lambda a, b: jnp.dot(a, b, preferred_element_type=jnp.float32)
    col_iota = jax.lax.broadcasted_iota(jnp.int32, (1, NPAD), 1)
    row_iota = jax.lax.broadcasted_iota(jnp.int32, (NPAD, 1), 0)
    row_ok = (row_iota < N_).astype(jnp.float32)

    P = Pr[...]
    G = Gr[...]
    C = Cr[...]
    Tcv = Tc[...]
    k1 = k1_ref[...]
    k2 = k2_ref[...]

    cn = jnp.sum(C * C, axis=1, keepdims=True)
    tnc = jnp.sum(Tcv * Tcv, axis=0, keepdims=True)
    e1sq = cn + tnc - 2.0 * dot(C, Tcv)

    i1 = k1 & IMASK
    i2 = k2 & IMASK
    d2v = lax.bitcast_convert_type(k2 & ~IMASK, jnp.float32)
    w2 = jnp.exp(-jnp.sqrt(jnp.maximum(d2v, 0.0))) * row_ok

    A1 = jnp.zeros((NPAD, NPAD), jnp.float32)
    A2 = jnp.zeros((NPAD, NPAD), jnp.float32)
    s1 = jnp.zeros((), jnp.float32)
    for t in range(NP_):
        m1 = (col_iota == i1[:, t:t + 1]).astype(jnp.float32)
        e1d = jnp.sum(m1 * e1sq, axis=1, keepdims=True)
        w1t = jnp.exp(-jnp.sqrt(jnp.maximum(e1d, 0.0))) * row_ok
        s1 = s1 + jnp.sum(w1t)
        A1 = A1 + w1t * m1
        m2 = (col_iota == i2[:, t:t + 1]).astype(jnp.float32)
        A2 = A2 + w2[:, t:t + 1] * m2
    r1 = 1.0 / (NP_ * s1)
    r2 = 1.0 / (NP_ * jnp.sum(w2))

    def mlp2(X, Wat, ba, Wbt, bb):
        return dot(_lrelu(dot(X, Wat[...]) + ba[...]), Wbt[...]) + bb[...]

    cf = mlp2(P, Wp1t, bp1, Wp2t, bp2)
    sfull = mlp2(G, W1t, b1, W2t, b2)
    spfull = mlp2(cf, Wps1t, bps1, Wps2t, bps2)

    sf = dot(A1, sfull) * r1
    sfp = dot(A2, spfull) * r2

    final1 = dot(sf, Wf1at[...]) + dot(cf, Wf1bt[...]) + bf1[...]
    final2 = dot(sfp, Wf2at[...]) + dot(G, Wf2bt[...]) + bf2[...]
    out_ref[...] = (dot(_lrelu(final2), Wfat[...])
                    + dot(_lrelu(final1), Wfbt[...]) + bf[...])


def kernel(img_feat, cloud, cloud_tar, W1, b1, W2, b2, Wps1, bps1, Wps2, bps2,
           Wp1, bp1, Wp2, bp2, Wf1, bf1, Wf2, bf2, Wf, bf):
    f32 = jnp.float32

    def pad1(v):
        return jnp.pad(v, (0, NPAD - N_), constant_values=1e17).astype(f32)

    def padr(x):
        return jnp.pad(x, ((0, NPAD - x.shape[0]), (0, 0))).astype(f32)

    P3 = cloud[0]
    T3 = cloud_tar[0]
    C2 = cloud.reshape(3, N_)

    k1x, k2x = _sc_knn(
        pad1(P3[:, 0]), pad1(P3[:, 1]), pad1(P3[:, 2]),
        pad1(T3[:, 0]), pad1(T3[:, 1]), pad1(T3[:, 2]))

    Pr = padr(jnp.pad(P3, ((0, 0), (0, 5))))
    Tr = padr(jnp.pad(T3, ((0, 0), (0, 5))))
    Cr = padr(jnp.pad(C2.T, ((0, 0), (0, 5))))
    Tc = Tr.T[:8]
    Gr = padr(img_feat[0].T)
    row2 = lambda b: b[None, :].astype(f32)

    tc_args = (
        Pr, Gr, Cr, Tc, _unlace(k1x), _unlace(k2x),
        jnp.pad(Wp1.T, ((0, 5), (0, 0))).astype(f32), row2(bp1),
        Wp2.T.astype(f32), row2(bp2),
        W1.T.astype(f32), row2(b1), W2.T.astype(f32), row2(b2),
        Wps1.T.astype(f32), row2(bps1), Wps2.T.astype(f32), row2(bps2),
        Wf1[:, :128].T.astype(f32), Wf1[:, 128:].T.astype(f32), row2(bf1),
        Wf2[:, :128].T.astype(f32), Wf2[:, 128:].T.astype(f32), row2(bf2),
        Wf[:, :64].T.astype(f32), Wf[:, 64:].T.astype(f32), row2(bf),
    )
    out = pl.pallas_call(
        _tc_body,
        out_shape=jax.ShapeDtypeStruct((NPAD, 64), f32),
        in_specs=[pl.BlockSpec(memory_space=pltpu.VMEM) for _ in tc_args],
        out_specs=pl.BlockSpec(memory_space=pltpu.VMEM),
    )(*tc_args)

    return out[:N_].T[None]
